# Initial kernel scaffold; baseline (speedup 1.0000x reference)
import jax
import jax.numpy as jnp
from jax.experimental import pallas as pl


def kernel(title, desc, t_len, d_len, table):
    raise NotImplementedError("write your pallas kernel here")


# speedup vs baseline: 3.9153x; 3.9153x over previous
import jax
import jax.numpy as jnp
from jax import lax
from jax.experimental import pallas as pl
from jax.experimental.pallas import tpu as pltpu
from jax.experimental.pallas import tpu_sc as plsc

B = 4096
LT = 20
LTP = 24
LD = 200
D = 32
H = 16
NC = 2
NS = 16
NW = NC * NS
RPW = B // NW
LTOT = LTP + LD


def _pool(rbuf, off, L):
    halves = []
    for h in (0, H):
        x0 = rbuf[off + 0, pl.ds(h, H)]
        x1 = rbuf[off + 1, pl.ds(h, H)]
        x2 = rbuf[off + 2, pl.ds(h, H)]
        x3 = rbuf[off + 3, pl.ds(h, H)]
        m3 = jnp.maximum(x0 + x1 + x2, x1 + x2 + x3)
        m5 = jnp.full((H,), -jnp.inf, jnp.float32)
        halves.append((x3, x2, x1, x0, m3, m5))

    def body(j, c):
        a1, a2, a3, a4, am3, am5, b1, b2, b3, b4, bm3, bm5 = c
        xa = rbuf[off + j, pl.ds(0, H)]
        xb = rbuf[off + j, pl.ds(H, H)]
        s3a = xa + a1 + a2
        s5a = s3a + a3 + a4
        s3b = xb + b1 + b2
        s5b = s3b + b3 + b4
        am3 = jnp.maximum(am3, s3a)
        am5 = jnp.maximum(am5, s5a)
        bm3 = jnp.maximum(bm3, s3b)
        bm5 = jnp.maximum(bm5, s5b)
        return (xa, a1, a2, a3, am3, am5, xb, b1, b2, b3, bm3, bm5)

    res = lax.fori_loop(4, L, body, halves[0] + halves[1])
    return res[4], res[10], res[5], res[11]


def _sc_body(title_hbm, desc_hbm, table_hbm, out_hbm,
             tidx, didx, rbuf0, rbuf1, obuf, sem0, sem1):
    cid = lax.axis_index("c")
    sid = lax.axis_index("s")
    wid = sid * NC + cid
    base = wid * RPW

    pltpu.sync_copy(title_hbm.at[pl.ds(base * LTP, RPW * LTP)], tidx)
    pltpu.sync_copy(desc_hbm.at[pl.ds(base * LD, RPW * LD)], didx)

    def issue(b, rbuf, sem):
        pltpu.async_copy(table_hbm.at[tidx.at[pl.ds(b * LTP, LTP)]],
                         rbuf.at[pl.ds(0, LTP)], sem)
        pltpu.async_copy(table_hbm.at[didx.at[pl.ds(b * LD, 128)]],
                         rbuf.at[pl.ds(LTP, 128)], sem)
        pltpu.async_copy(table_hbm.at[didx.at[pl.ds(b * LD + 128, 72)]],
                         rbuf.at[pl.ds(LTP + 128, 72)], sem)

    def drain(rbuf, sem):
        pltpu.make_async_copy(table_hbm.at[pl.ds(0, LTOT)], rbuf, sem).wait()

    def compute(b, rbuf):
        t3l, t3h, t5l, t5h = _pool(rbuf, 0, LT)
        d3l, d3h, d5l, d5h = _pool(rbuf, LTP, LD)
        obuf[b, pl.ds(0, H)] = t3l / 3.0
        obuf[b, pl.ds(H, H)] = t3h / 3.0
        obuf[b, pl.ds(D, H)] = d3l / 3.0
        obuf[b, pl.ds(D + H, H)] = d3h / 3.0
        obuf[b, pl.ds(2 * D, H)] = t5l / 5.0
        obuf[b, pl.ds(2 * D + H, H)] = t5h / 5.0
        obuf[b, pl.ds(3 * D, H)] = d5l / 5.0
        obuf[b, pl.ds(3 * D + H, H)] = d5h / 5.0

    issue(0, rbuf0, sem0)

    def rowpair(p, carry):
        b0 = p * 2
        issue(b0 + 1, rbuf1, sem1)
        drain(rbuf0, sem0)
        compute(b0, rbuf0)

        @pl.when(b0 + 2 < RPW)
        def _():
            issue(b0 + 2, rbuf0, sem0)

        drain(rbuf1, sem1)
        compute(b0 + 1, rbuf1)
        return carry

    lax.fori_loop(0, RPW // 2, rowpair, 0)

    pltpu.sync_copy(obuf, out_hbm.at[pl.ds(base, RPW)])


@jax.jit
def kernel(title, desc, t_len, d_len, table):
    del t_len, d_len
    title_p = jnp.pad(title, ((0, 0), (0, LTP - LT))).reshape(-1)
    desc_f = desc.reshape(-1)
    mesh = plsc.VectorSubcoreMesh(core_axis_name="c", subcore_axis_name="s")
    run = pl.kernel(
        _sc_body,
        mesh=mesh,
        compiler_params=pltpu.CompilerParams(use_tc_tiling_on_sc=False),
        out_type=jax.ShapeDtypeStruct((B, 4 * D), jnp.float32),
        scratch_types=[
            pltpu.VMEM((RPW * LTP,), jnp.int32),
            pltpu.VMEM((RPW * LD,), jnp.int32),
            pltpu.VMEM((LTOT, D), jnp.float32),
            pltpu.VMEM((LTOT, D), jnp.float32),
            pltpu.VMEM((RPW, 4 * D), jnp.float32),
            pltpu.SemaphoreType.DMA,
            pltpu.SemaphoreType.DMA,
        ],
    )
    return run(title_p, desc_f, table)


# speedup vs baseline: 3.9302x; 1.0038x over previous
import jax
import jax.numpy as jnp
from jax import lax
from jax.experimental import pallas as pl
from jax.experimental.pallas import tpu as pltpu
from jax.experimental.pallas import tpu_sc as plsc

B = 4096
LT = 20
LD = 200
D = 32
H = 16
NC = 2
NS = 16
NW = NC * NS
RPW = B // NW
NP = RPW // 2
LPAIR = 2 * LT + 2 * LD
DO0 = 2 * LT
DO1 = 2 * LT + LD


def _pool(rbuf, off, L):
    halves = []
    for h in (0, H):
        x0 = rbuf[off + 0, pl.ds(h, H)]
        x1 = rbuf[off + 1, pl.ds(h, H)]
        x2 = rbuf[off + 2, pl.ds(h, H)]
        x3 = rbuf[off + 3, pl.ds(h, H)]
        m3 = jnp.maximum(x0 + x1 + x2, x1 + x2 + x3)
        m5 = jnp.full((H,), -jnp.inf, jnp.float32)
        halves.append((x3, x2, x1, x0, m3, m5))

    def body(j, c):
        a1, a2, a3, a4, am3, am5, b1, b2, b3, b4, bm3, bm5 = c
        xa = rbuf[off + j, pl.ds(0, H)]
        xb = rbuf[off + j, pl.ds(H, H)]
        s3a = xa + a1 + a2
        s5a = s3a + a3 + a4
        s3b = xb + b1 + b2
        s5b = s3b + b3 + b4
        am3 = jnp.maximum(am3, s3a)
        am5 = jnp.maximum(am5, s5a)
        bm3 = jnp.maximum(bm3, s3b)
        bm5 = jnp.maximum(bm5, s5b)
        return (xa, a1, a2, a3, am3, am5, xb, b1, b2, b3, bm3, bm5)

    res = lax.fori_loop(4, L, body, halves[0] + halves[1])
    return res[4], res[10], res[5], res[11]


def _sc_body(title_hbm, desc_hbm, table_hbm, out_hbm,
             tidx, didx, rbuf0, rbuf1, obuf, sem0, sem1):
    cid = lax.axis_index("c")
    sid = lax.axis_index("s")
    wid = sid * NC + cid
    base = wid * RPW

    pltpu.sync_copy(title_hbm.at[pl.ds(base * LT, RPW * LT)], tidx)
    pltpu.sync_copy(desc_hbm.at[pl.ds(base * LD, RPW * LD)], didx)

    def issue(p, rbuf, sem):
        pltpu.async_copy(table_hbm.at[tidx.at[pl.ds(p * 2 * LT, 2 * LT)]],
                         rbuf.at[pl.ds(0, 2 * LT)], sem)
        pltpu.async_copy(table_hbm.at[didx.at[pl.ds(p * 2 * LD, 128)]],
                         rbuf.at[pl.ds(DO0, 128)], sem)
        pltpu.async_copy(table_hbm.at[didx.at[pl.ds(p * 2 * LD + 128, 72)]],
                         rbuf.at[pl.ds(DO0 + 128, 72)], sem)
        pltpu.async_copy(table_hbm.at[didx.at[pl.ds(p * 2 * LD + LD, 128)]],
                         rbuf.at[pl.ds(DO1, 128)], sem)
        pltpu.async_copy(table_hbm.at[didx.at[pl.ds(p * 2 * LD + LD + 128, 72)]],
                         rbuf.at[pl.ds(DO1 + 128, 72)], sem)

    def drain(rbuf, sem):
        pltpu.make_async_copy(table_hbm.at[pl.ds(0, LPAIR)], rbuf, sem).wait()

    def compute(p, rbuf):
        for r, toff, doff in ((0, 0, DO0), (1, LT, DO1)):
            b = p * 2 + r
            t3l, t3h, t5l, t5h = _pool(rbuf, toff, LT)
            d3l, d3h, d5l, d5h = _pool(rbuf, doff, LD)
            obuf[b, pl.ds(0, H)] = t3l / 3.0
            obuf[b, pl.ds(H, H)] = t3h / 3.0
            obuf[b, pl.ds(D, H)] = d3l / 3.0
            obuf[b, pl.ds(D + H, H)] = d3h / 3.0
            obuf[b, pl.ds(2 * D, H)] = t5l / 5.0
            obuf[b, pl.ds(2 * D + H, H)] = t5h / 5.0
            obuf[b, pl.ds(3 * D, H)] = d5l / 5.0
            obuf[b, pl.ds(3 * D + H, H)] = d5h / 5.0

    issue(0, rbuf0, sem0)

    def pairpair(i, carry):
        p0 = i * 2
        issue(p0 + 1, rbuf1, sem1)
        drain(rbuf0, sem0)
        compute(p0, rbuf0)

        @pl.when(p0 + 2 < NP)
        def _():
            issue(p0 + 2, rbuf0, sem0)

        drain(rbuf1, sem1)
        compute(p0 + 1, rbuf1)
        return carry

    lax.fori_loop(0, NP // 2, pairpair, 0)

    pltpu.sync_copy(obuf, out_hbm.at[pl.ds(base, RPW)])


@jax.jit
def kernel(title, desc, t_len, d_len, table):
    del t_len, d_len
    mesh = plsc.VectorSubcoreMesh(core_axis_name="c", subcore_axis_name="s")
    run = pl.kernel(
        _sc_body,
        mesh=mesh,
        compiler_params=pltpu.CompilerParams(use_tc_tiling_on_sc=False),
        out_type=jax.ShapeDtypeStruct((B, 4 * D), jnp.float32),
        scratch_types=[
            pltpu.VMEM((RPW * LT,), jnp.int32),
            pltpu.VMEM((RPW * LD,), jnp.int32),
            pltpu.VMEM((LPAIR, D), jnp.float32),
            pltpu.VMEM((LPAIR, D), jnp.float32),
            pltpu.VMEM((RPW, 4 * D), jnp.float32),
            pltpu.SemaphoreType.DMA,
            pltpu.SemaphoreType.DMA,
        ],
    )
    return run(title.reshape(-1), desc.reshape(-1), table)


# speedup vs baseline: 5.0801x; 1.2926x over previous
import jax
import jax.numpy as jnp
from jax import lax
from jax.experimental import pallas as pl
from jax.experimental.pallas import tpu as pltpu
from jax.experimental.pallas import tpu_sc as plsc

B = 4096
LT = 20
LD = 200
D = 32
H = 16
NC = 2
NS = 16
NW = NC * NS
RPW = B // NW
NP = RPW // 2
LPAIR = 2 * LT + 2 * LD
DO0 = 2 * LT
DO1 = 2 * LT + LD


def _pool(rbuf, off, L):
    halves = []
    for h in (0, H):
        x0 = rbuf[off + 0, pl.ds(h, H)]
        x1 = rbuf[off + 1, pl.ds(h, H)]
        x2 = rbuf[off + 2, pl.ds(h, H)]
        x3 = rbuf[off + 3, pl.ds(h, H)]
        m3 = jnp.maximum(x0 + x1 + x2, x1 + x2 + x3)
        m5 = jnp.full((H,), -jnp.inf, jnp.float32)
        halves.append((x3, x2, x1, x0, m3, m5))

    def upd(x, r1, r2, r3, r4, m3, m5):
        s30 = x[0] + r1 + r2
        s50 = s30 + r3 + r4
        s31 = x[1] + x[0] + r1
        s51 = s31 + r2 + r3
        s32 = x[2] + x[1] + x[0]
        s52 = s32 + r1 + r2
        s33 = x[3] + x[2] + x[1]
        s53 = s33 + x[0] + r1
        m3 = jnp.maximum(m3, jnp.maximum(jnp.maximum(s30, s31),
                                         jnp.maximum(s32, s33)))
        m5 = jnp.maximum(m5, jnp.maximum(jnp.maximum(s50, s51),
                                         jnp.maximum(s52, s53)))
        return (x[3], x[2], x[1], x[0], m3, m5)

    def body(i, c):
        a1, a2, a3, a4, am3, am5, b1, b2, b3, b4, bm3, bm5 = c
        base = off + 4 + i * 4
        xa = [rbuf[base + t, pl.ds(0, H)] for t in range(4)]
        xb = [rbuf[base + t, pl.ds(H, H)] for t in range(4)]
        return (upd(xa, a1, a2, a3, a4, am3, am5)
                + upd(xb, b1, b2, b3, b4, bm3, bm5))

    res = lax.fori_loop(0, (L - 4) // 4, body, halves[0] + halves[1])
    return res[4], res[10], res[5], res[11]


def _sc_body(title_hbm, desc_hbm, table_hbm, out_hbm,
             tidx, didx, rbuf0, rbuf1, obuf, sem0, sem1):
    cid = lax.axis_index("c")
    sid = lax.axis_index("s")
    wid = sid * NC + cid
    base = wid * RPW

    pltpu.sync_copy(title_hbm.at[pl.ds(base * LT, RPW * LT)], tidx)
    pltpu.sync_copy(desc_hbm.at[pl.ds(base * LD, RPW * LD)], didx)

    def issue(p, rbuf, sem):
        pltpu.async_copy(table_hbm.at[tidx.at[pl.ds(p * 2 * LT, 2 * LT)]],
                         rbuf.at[pl.ds(0, 2 * LT)], sem)
        pltpu.async_copy(table_hbm.at[didx.at[pl.ds(p * 2 * LD, 128)]],
                         rbuf.at[pl.ds(DO0, 128)], sem)
        pltpu.async_copy(table_hbm.at[didx.at[pl.ds(p * 2 * LD + 128, 72)]],
                         rbuf.at[pl.ds(DO0 + 128, 72)], sem)
        pltpu.async_copy(table_hbm.at[didx.at[pl.ds(p * 2 * LD + LD, 128)]],
                         rbuf.at[pl.ds(DO1, 128)], sem)
        pltpu.async_copy(table_hbm.at[didx.at[pl.ds(p * 2 * LD + LD + 128, 72)]],
                         rbuf.at[pl.ds(DO1 + 128, 72)], sem)

    def drain(rbuf, sem):
        pltpu.make_async_copy(table_hbm.at[pl.ds(0, LPAIR)], rbuf, sem).wait()

    def compute(p, rbuf):
        for r, toff, doff in ((0, 0, DO0), (1, LT, DO1)):
            b = p * 2 + r
            t3l, t3h, t5l, t5h = _pool(rbuf, toff, LT)
            d3l, d3h, d5l, d5h = _pool(rbuf, doff, LD)
            obuf[b, pl.ds(0, H)] = t3l / 3.0
            obuf[b, pl.ds(H, H)] = t3h / 3.0
            obuf[b, pl.ds(D, H)] = d3l / 3.0
            obuf[b, pl.ds(D + H, H)] = d3h / 3.0
            obuf[b, pl.ds(2 * D, H)] = t5l / 5.0
            obuf[b, pl.ds(2 * D + H, H)] = t5h / 5.0
            obuf[b, pl.ds(3 * D, H)] = d5l / 5.0
            obuf[b, pl.ds(3 * D + H, H)] = d5h / 5.0

    issue(0, rbuf0, sem0)

    def pairpair(i, carry):
        p0 = i * 2
        issue(p0 + 1, rbuf1, sem1)
        drain(rbuf0, sem0)
        compute(p0, rbuf0)

        @pl.when(p0 + 2 < NP)
        def _():
            issue(p0 + 2, rbuf0, sem0)

        drain(rbuf1, sem1)
        compute(p0 + 1, rbuf1)
        return carry

    lax.fori_loop(0, NP // 2, pairpair, 0)

    pltpu.sync_copy(obuf, out_hbm.at[pl.ds(base, RPW)])


@jax.jit
def kernel(title, desc, t_len, d_len, table):
    del t_len, d_len
    mesh = plsc.VectorSubcoreMesh(core_axis_name="c", subcore_axis_name="s")
    run = pl.kernel(
        _sc_body,
        mesh=mesh,
        compiler_params=pltpu.CompilerParams(use_tc_tiling_on_sc=False),
        out_type=jax.ShapeDtypeStruct((B, 4 * D), jnp.float32),
        scratch_types=[
            pltpu.VMEM((RPW * LT,), jnp.int32),
            pltpu.VMEM((RPW * LD,), jnp.int32),
            pltpu.VMEM((LPAIR, D), jnp.float32),
            pltpu.VMEM((LPAIR, D), jnp.float32),
            pltpu.VMEM((RPW, 4 * D), jnp.float32),
            pltpu.SemaphoreType.DMA,
            pltpu.SemaphoreType.DMA,
        ],
    )
    return run(title.reshape(-1), desc.reshape(-1), table)
